# Initial kernel scaffold; baseline (speedup 1.0000x reference)
import jax
import jax.numpy as jnp
from jax.experimental import pallas as pl


def kernel(X, edge_index, p, W_ih, W_hh, b_ih, b_hh, initial_weight):
    raise NotImplementedError("write your pallas kernel here")


# speedup vs baseline: 1.4949x; 1.4949x over previous
import jax
import jax.numpy as jnp
from jax.experimental import pallas as pl
from jax.experimental.pallas import tpu as pltpu

_ROW_BLK = 2000
_EDGE_BLK = 1280


def _score_kernel(x_ref, p_ref, s_ref):
    pv = p_ref[...]
    inv = 1.0 / (jnp.sqrt(jnp.sum(pv * pv)) + 1e-16)
    s_ref[...] = jnp.tanh(
        jnp.dot(x_ref[...], pv, preferred_element_type=jnp.float32) * inv)


def _gru_kernel(perm_ref, ts_ref, x_ref, wih_ref, whh_ref, bih_ref, bhh_ref,
                h_ref, w_ref, xt_ref):
    k = xt_ref.shape[0]
    c = h_ref.shape[0]

    def gather(i, carry):
        r = perm_ref[0, i]
        row = x_ref[pl.ds(r, 1), :]
        s = ts_ref[pl.ds(i, 1), :]
        xt_ref[pl.ds(i, 1), :] = row * s
        return carry

    jax.lax.fori_loop(0, k, gather, 0)
    xt = xt_ref[...]
    gi = jnp.dot(xt, wih_ref[...], preferred_element_type=jnp.float32) \
        + bih_ref[...]
    gh = jnp.dot(h_ref[...], whh_ref[...], preferred_element_type=jnp.float32) \
        + bhh_ref[...]
    r = jax.nn.sigmoid(gi[:, :c] + gh[:, :c])
    z = jax.nn.sigmoid(gi[:, c:2 * c] + gh[:, c:2 * c])
    n = jnp.tanh(gi[:, 2 * c:] + r * gh[:, 2 * c:])
    w_ref[...] = (1.0 - z) * n + z * h_ref[...]


def _hist_kernel(col_ref, deg_ref):
    @pl.when(pl.program_id(0) == 0)
    def _():
        deg_ref[...] = jnp.ones_like(deg_ref)

    def body(j, carry):
        cidx = col_ref[0, j]
        deg_ref[pl.ds(cidx, 1), :] = deg_ref[pl.ds(cidx, 1), :] + 1.0
        return carry

    jax.lax.fori_loop(0, col_ref.shape[1], body, 0)


def _y_kernel(x_ref, w_ref, deg_ref, y_ref, dinv_ref):
    deg = deg_ref[...]
    dinv = jnp.where(deg > 0, jax.lax.rsqrt(deg), 0.0)
    dinv_ref[...] = dinv
    y_ref[...] = dinv * jnp.dot(x_ref[...], w_ref[...],
                                preferred_element_type=jnp.float32)


def _scatter_kernel(e_ref, y_ref, dinv_ref, out_ref):
    @pl.when(pl.program_id(0) == 0)
    def _():
        out_ref[...] = y_ref[...]

    def body(j, carry):
        r = e_ref[0, j]
        cidx = e_ref[1, j]
        out_ref[pl.ds(cidx, 1), :] = out_ref[pl.ds(cidx, 1), :] + y_ref[pl.ds(r, 1), :]
        return carry

    jax.lax.fori_loop(0, e_ref.shape[1], body, 0)

    @pl.when(pl.program_id(0) == pl.num_programs(0) - 1)
    def _():
        out_ref[...] = out_ref[...] * dinv_ref[...]


def kernel(X, edge_index, p, W_ih, W_hh, b_ih, b_hh, initial_weight):
    n, c = X.shape
    e = edge_index.shape[1]
    k = initial_weight.shape[0]
    seq = pltpu.CompilerParams(dimension_semantics=("arbitrary",))

    score = pl.pallas_call(
        _score_kernel,
        grid=(n // _ROW_BLK,),
        in_specs=[
            pl.BlockSpec((_ROW_BLK, c), lambda i: (i, 0)),
            pl.BlockSpec((c, 1), lambda i: (0, 0)),
        ],
        out_specs=pl.BlockSpec((_ROW_BLK, 1), lambda i: (i, 0)),
        out_shape=jax.ShapeDtypeStruct((n, 1), jnp.float32),
        compiler_params=seq,
    )(X, p[:, None]).reshape(n)

    top_scores, perm = jax.lax.top_k(score, k)
    W = pl.pallas_call(
        _gru_kernel,
        grid=(1,),
        in_specs=[
            pl.BlockSpec((1, k), lambda i: (0, 0), memory_space=pltpu.SMEM),
            pl.BlockSpec((k, 1), lambda i: (0, 0)),
            pl.BlockSpec((n, c), lambda i: (0, 0)),
            pl.BlockSpec((c, 3 * c), lambda i: (0, 0)),
            pl.BlockSpec((c, 3 * c), lambda i: (0, 0)),
            pl.BlockSpec((1, 3 * c), lambda i: (0, 0)),
            pl.BlockSpec((1, 3 * c), lambda i: (0, 0)),
            pl.BlockSpec((c, c), lambda i: (0, 0)),
        ],
        out_specs=pl.BlockSpec((c, c), lambda i: (0, 0)),
        out_shape=jax.ShapeDtypeStruct((c, c), jnp.float32),
        scratch_shapes=[pltpu.VMEM((k, c), jnp.float32)],
        compiler_params=seq,
    )(perm[None, :], top_scores[:, None], X, W_ih.T, W_hh.T,
      b_ih[None, :], b_hh[None, :], initial_weight)

    deg = pl.pallas_call(
        _hist_kernel,
        grid=(e // _EDGE_BLK,),
        in_specs=[
            pl.BlockSpec((1, _EDGE_BLK), lambda i: (0, i),
                         memory_space=pltpu.SMEM),
        ],
        out_specs=pl.BlockSpec((n, 1), lambda i: (0, 0)),
        out_shape=jax.ShapeDtypeStruct((n, 1), jnp.float32),
        compiler_params=seq,
    )(edge_index[1][None, :])

    Y, dinv = pl.pallas_call(
        _y_kernel,
        grid=(n // _ROW_BLK,),
        in_specs=[
            pl.BlockSpec((_ROW_BLK, c), lambda i: (i, 0)),
            pl.BlockSpec((c, c), lambda i: (0, 0)),
            pl.BlockSpec((_ROW_BLK, 1), lambda i: (i, 0)),
        ],
        out_specs=[
            pl.BlockSpec((_ROW_BLK, c), lambda i: (i, 0)),
            pl.BlockSpec((_ROW_BLK, 1), lambda i: (i, 0)),
        ],
        out_shape=[
            jax.ShapeDtypeStruct((n, c), jnp.float32),
            jax.ShapeDtypeStruct((n, 1), jnp.float32),
        ],
        compiler_params=seq,
    )(X, W, deg)

    out = pl.pallas_call(
        _scatter_kernel,
        grid=(e // _EDGE_BLK,),
        in_specs=[
            pl.BlockSpec((2, _EDGE_BLK), lambda i: (0, i),
                         memory_space=pltpu.SMEM),
            pl.BlockSpec((n, c), lambda i: (0, 0)),
            pl.BlockSpec((n, 1), lambda i: (0, 0)),
        ],
        out_specs=pl.BlockSpec((n, c), lambda i: (0, 0)),
        out_shape=jax.ShapeDtypeStruct((n, c), jnp.float32),
        compiler_params=seq,
    )(edge_index, Y, dinv)
    return out


# speedup vs baseline: 2.9791x; 1.9928x over previous
import jax
import jax.numpy as jnp
from jax.experimental import pallas as pl
from jax.experimental.pallas import tpu as pltpu

_ROW_BLK = 2000
_EDGE_BLK = 1280


def _score_kernel(x_ref, p_ref, s_ref):
    pv = p_ref[...]
    inv = 1.0 / (jnp.sqrt(jnp.sum(pv * pv)) + 1e-16)
    s_ref[...] = jnp.tanh(
        jnp.dot(x_ref[...], pv, preferred_element_type=jnp.float32) * inv)


def _gru_kernel(perm_ref, ts_ref, x_ref, wih_ref, whh_ref, bih_ref, bhh_ref,
                h_ref, w_ref, xt_ref):
    k = xt_ref.shape[0]
    c = h_ref.shape[0]

    def gather(i, carry):
        r = perm_ref[0, i]
        row = x_ref[pl.ds(r, 1), :]
        s = ts_ref[pl.ds(i, 1), :]
        xt_ref[pl.ds(i, 1), :] = row * s
        return carry

    jax.lax.fori_loop(0, k, gather, 0)
    xt = xt_ref[...]
    gi = jnp.dot(xt, wih_ref[...], preferred_element_type=jnp.float32) \
        + bih_ref[...]
    gh = jnp.dot(h_ref[...], whh_ref[...], preferred_element_type=jnp.float32) \
        + bhh_ref[...]
    r = jax.nn.sigmoid(gi[:, :c] + gh[:, :c])
    z = jax.nn.sigmoid(gi[:, c:2 * c] + gh[:, c:2 * c])
    n = jnp.tanh(gi[:, 2 * c:] + r * gh[:, 2 * c:])
    w_ref[...] = (1.0 - z) * n + z * h_ref[...]


def _hist_kernel(col_ref, deg_ref):
    @pl.when(pl.program_id(0) == 0)
    def _():
        deg_ref[...] = jnp.ones_like(deg_ref)

    def body(j, carry):
        cidx = col_ref[0, j]
        deg_ref[pl.ds(cidx, 1), :] = deg_ref[pl.ds(cidx, 1), :] + 1.0
        return carry

    jax.lax.fori_loop(0, col_ref.shape[1], body, 0, unroll=8)


def _y_kernel(x_ref, w_ref, deg_ref, y_ref, dinv_ref):
    deg = deg_ref[...]
    dinv = jnp.where(deg > 0, jax.lax.rsqrt(deg), 0.0)
    dinv_ref[...] = dinv
    y_ref[...] = dinv * jnp.dot(x_ref[...], w_ref[...],
                                preferred_element_type=jnp.float32)


def _scatter_kernel(e_ref, y_ref, dinv_ref, out_ref):
    @pl.when(pl.program_id(0) == 0)
    def _():
        out_ref[...] = y_ref[...]

    def body(j, carry):
        r = e_ref[0, j]
        cidx = e_ref[1, j]
        out_ref[pl.ds(cidx, 1), :] = out_ref[pl.ds(cidx, 1), :] + y_ref[pl.ds(r, 1), :]
        return carry

    jax.lax.fori_loop(0, e_ref.shape[1], body, 0, unroll=8)

    @pl.when(pl.program_id(0) == pl.num_programs(0) - 1)
    def _():
        out_ref[...] = out_ref[...] * dinv_ref[...]


def kernel(X, edge_index, p, W_ih, W_hh, b_ih, b_hh, initial_weight):
    n, c = X.shape
    e = edge_index.shape[1]
    k = initial_weight.shape[0]
    seq = pltpu.CompilerParams(dimension_semantics=("arbitrary",))

    score = pl.pallas_call(
        _score_kernel,
        grid=(n // _ROW_BLK,),
        in_specs=[
            pl.BlockSpec((_ROW_BLK, c), lambda i: (i, 0)),
            pl.BlockSpec((c, 1), lambda i: (0, 0)),
        ],
        out_specs=pl.BlockSpec((_ROW_BLK, 1), lambda i: (i, 0)),
        out_shape=jax.ShapeDtypeStruct((n, 1), jnp.float32),
        compiler_params=seq,
    )(X, p[:, None]).reshape(n)

    top_scores, perm = jax.lax.top_k(score, k)
    W = pl.pallas_call(
        _gru_kernel,
        grid=(1,),
        in_specs=[
            pl.BlockSpec((1, k), lambda i: (0, 0), memory_space=pltpu.SMEM),
            pl.BlockSpec((k, 1), lambda i: (0, 0)),
            pl.BlockSpec((n, c), lambda i: (0, 0)),
            pl.BlockSpec((c, 3 * c), lambda i: (0, 0)),
            pl.BlockSpec((c, 3 * c), lambda i: (0, 0)),
            pl.BlockSpec((1, 3 * c), lambda i: (0, 0)),
            pl.BlockSpec((1, 3 * c), lambda i: (0, 0)),
            pl.BlockSpec((c, c), lambda i: (0, 0)),
        ],
        out_specs=pl.BlockSpec((c, c), lambda i: (0, 0)),
        out_shape=jax.ShapeDtypeStruct((c, c), jnp.float32),
        scratch_shapes=[pltpu.VMEM((k, c), jnp.float32)],
        compiler_params=seq,
    )(perm[None, :], top_scores[:, None], X, W_ih.T, W_hh.T,
      b_ih[None, :], b_hh[None, :], initial_weight)

    deg = pl.pallas_call(
        _hist_kernel,
        grid=(e // _EDGE_BLK,),
        in_specs=[
            pl.BlockSpec((1, _EDGE_BLK), lambda i: (0, i),
                         memory_space=pltpu.SMEM),
        ],
        out_specs=pl.BlockSpec((n, 1), lambda i: (0, 0)),
        out_shape=jax.ShapeDtypeStruct((n, 1), jnp.float32),
        compiler_params=seq,
    )(edge_index[1][None, :])

    Y, dinv = pl.pallas_call(
        _y_kernel,
        grid=(n // _ROW_BLK,),
        in_specs=[
            pl.BlockSpec((_ROW_BLK, c), lambda i: (i, 0)),
            pl.BlockSpec((c, c), lambda i: (0, 0)),
            pl.BlockSpec((_ROW_BLK, 1), lambda i: (i, 0)),
        ],
        out_specs=[
            pl.BlockSpec((_ROW_BLK, c), lambda i: (i, 0)),
            pl.BlockSpec((_ROW_BLK, 1), lambda i: (i, 0)),
        ],
        out_shape=[
            jax.ShapeDtypeStruct((n, c), jnp.float32),
            jax.ShapeDtypeStruct((n, 1), jnp.float32),
        ],
        compiler_params=seq,
    )(X, W, deg)

    out = pl.pallas_call(
        _scatter_kernel,
        grid=(e // _EDGE_BLK,),
        in_specs=[
            pl.BlockSpec((2, _EDGE_BLK), lambda i: (0, i),
                         memory_space=pltpu.SMEM),
            pl.BlockSpec((n, c), lambda i: (0, 0)),
            pl.BlockSpec((n, 1), lambda i: (0, 0)),
        ],
        out_specs=pl.BlockSpec((n, c), lambda i: (0, 0)),
        out_shape=jax.ShapeDtypeStruct((n, c), jnp.float32),
        compiler_params=seq,
    )(edge_index, Y, dinv)
    return out
